# Initial kernel scaffold; baseline (speedup 1.0000x reference)
import jax
import jax.numpy as jnp
from jax.experimental import pallas as pl


def kernel(x, active_user, edge_index, edge_weight, interact_user_idx, interact_poi_idx, interact_weight, enc_table, user_table, fc_W, fc_b):
    raise NotImplementedError("write your pallas kernel here")


# speedup vs baseline: 6.0944x; 6.0944x over previous
import functools

import jax
import jax.numpy as jnp
from jax import lax
from jax.experimental import pallas as pl
from jax.experimental.pallas import tpu as pltpu
from jax.experimental.pallas import tpu_sc as plsc

P_TOT = 10000
U_TOT = 5000
H = 128
S = 20
B = 128
NSLOT = S * B
DUMMY = NSLOT
ACC_ROWS = 2688
UACC_ROWS = 256
UDUMMY = B

N_EDGES = 320000
N_INTER = 200000
NTILES = 16
E1T = 20096
EIT = 12544
E1_PAD = E1T * NTILES
EI_PAD = EIT * NTILES
STAGE1 = E1T // 4
STAGEI = EIT // 4
CSBUF = STAGE1 + 128

_i32 = jnp.int32
_f32 = jnp.float32


def _sc_body(x_hbm, au_hbm, e1s_hbm, e1d_hbm, e1w_hbm, eiu_hbm, eip_hbm,
             eiw_hbm, enc_hbm, usr_hbm,
             enc_out, poi_out, uacc_out, r_out, ru_out, pu_out,
             poi_map, usr_map, x_v, au_v, st_g, st_f, st_w,
             cs_src, cs_w, cs_slot, idx2d, rows_v, r_v, enc_sh, poi_sh,
             u_sh, sem):
    c = lax.axis_index("c")
    s = lax.axis_index("s")
    zero16 = jnp.zeros((16,), _f32)
    neg16 = jnp.full((16,), -1, _i32)

    pltpu.sync_copy(x_hbm, x_v)
    pltpu.sync_copy(au_hbm, au_v)

    def init_poi(i, _):
        poi_map[pl.ds(i * 16, 16)] = neg16
        return 0
    lax.fori_loop(0, P_TOT // 16, init_poi, 0)

    def init_usr(i, _):
        usr_map[pl.ds(i * 16, 16)] = neg16
        return 0
    lax.fori_loop(0, 5008 // 16, init_usr, 0)

    iota16 = lax.iota(_i32, 16)

    def scat_poi(i, _):
        idx = x_v[pl.ds(i * 16, 16)]
        plsc.store_scatter(poi_map, [idx], i * 16 + iota16)
        return 0
    lax.fori_loop(0, NSLOT // 16, scat_poi, 0)

    def scat_usr(i, _):
        idx = au_v[pl.ds(i * 16, 16)]
        plsc.store_scatter(usr_map, [idx], i * 16 + iota16)
        return 0
    lax.fori_loop(0, B // 16, scat_usr, 0)

    def zrow(i, _):
        for k in range(8):
            rows_v[i, pl.ds(k * 16, 16)] = zero16
        return 0
    lax.fori_loop(0, 128, zrow, 0)

    stripe = ACC_ROWS // NTILES

    @pl.when(c == 0)
    def _():
        pltpu.sync_copy(rows_v, enc_sh.at[pl.ds(s * stripe, 128)])
        pltpu.sync_copy(rows_v.at[pl.ds(0, stripe - 128)],
                        enc_sh.at[pl.ds(s * stripe + 128, stripe - 128)])

    @pl.when(c == 1)
    def _():
        pltpu.sync_copy(rows_v, poi_sh.at[pl.ds(s * stripe, 128)])
        pltpu.sync_copy(rows_v.at[pl.ds(0, stripe - 128)],
                        poi_sh.at[pl.ds(s * stripe + 128, stripe - 128)])
        pltpu.sync_copy(rows_v.at[pl.ds(0, UACC_ROWS // NTILES)],
                        u_sh.at[pl.ds(s * (UACC_ROWS // NTILES),
                                      UACC_ROWS // NTILES)])

    plsc.subcore_barrier()

    def run_phase(gsrc_hbm, fdst_hbm, w_hbm, per_tile, stage_sz, map_ref,
                  table_hbm, acc_sh, dummy_slot):
        base = s * per_tile
        dum16 = jnp.full((16,), dummy_slot, _i32)
        zi16 = jnp.zeros((16,), _i32)

        for st in range(4):
            off = base + st * stage_sz
            pltpu.sync_copy(gsrc_hbm.at[pl.ds(off, stage_sz)],
                            st_g.at[pl.ds(0, stage_sz)])
            pltpu.sync_copy(fdst_hbm.at[pl.ds(off, stage_sz)],
                            st_f.at[pl.ds(0, stage_sz)])
            pltpu.sync_copy(w_hbm.at[pl.ds(off, stage_sz)],
                            st_w.at[pl.ds(0, stage_sz)])

            def fbody(i, cnt):
                fv = st_f[pl.ds(i * 16, 16)]
                slot = plsc.load_gather(map_ref, [fv])
                m = slot >= 0
                plsc.store_compressed(cs_src.at[pl.ds(cnt, 16)],
                                      st_g[pl.ds(i * 16, 16)], mask=m)
                plsc.store_compressed(cs_w.at[pl.ds(cnt, 16)],
                                      st_w[pl.ds(i * 16, 16)], mask=m)
                plsc.store_compressed(cs_slot.at[pl.ds(cnt, 16)], slot,
                                      mask=m)
                return cnt + jnp.sum(m.astype(_i32))
            cnt = lax.fori_loop(0, stage_sz // 16, fbody, jnp.int32(0))

            for k in range(8):
                cs_src[pl.ds(cnt + k * 16, 16)] = zi16
                cs_w[pl.ds(cnt + k * 16, 16)] = zero16
                cs_slot[pl.ds(cnt + k * 16, 16)] = dum16
            nsc = (cnt + 127) // 128

            def gbody(j, _):
                for k in range(8):
                    idx2d[0, pl.ds(k * 16, 16)] = (
                        cs_slot[pl.ds(j * 128 + k * 16, 16)])
                pltpu.async_copy(
                    table_hbm.at[cs_src.at[pl.ds(j * 128, 128)]],
                    rows_v, sem).wait()

                def rbody(g, _):
                    wv = cs_w[pl.ds(j * 128 + g * 16, 16)]
                    for i in range(16):
                        w = wv[i]
                        row = g * 16 + i
                        for k in range(8):
                            sl = pl.ds(k * 16, 16)
                            rows_v[row, sl] = rows_v[row, sl] * w
                    return 0
                lax.fori_loop(0, 8, rbody, 0)
                pltpu.sync_copy(rows_v, acc_sh.at[idx2d.at[0]], add=True)
                return 0
            lax.fori_loop(0, nsc, gbody, 0)

    @pl.when(c == 0)
    def _():
        run_phase(e1s_hbm, e1d_hbm, e1w_hbm, E1T, STAGE1, poi_map,
                  enc_hbm, enc_sh, DUMMY)

    @pl.when(c == 1)
    def _():
        run_phase(eiu_hbm, eip_hbm, eiw_hbm, EIT, STAGEI, poi_map,
                  usr_hbm, poi_sh, DUMMY)
        run_phase(eip_hbm, eiu_hbm, eiw_hbm, EIT, STAGEI, usr_map,
                  enc_hbm, u_sh, UDUMMY)

    plsc.subcore_barrier()

    @pl.when(c == 0)
    def _():
        pltpu.sync_copy(enc_sh.at[pl.ds(s * stripe, stripe)],
                        enc_out.at[pl.ds(s * stripe, stripe)])

    @pl.when(c == 1)
    def _():
        pltpu.sync_copy(poi_sh.at[pl.ds(s * stripe, stripe)],
                        poi_out.at[pl.ds(s * stripe, stripe)])
        ustripe = UACC_ROWS // NTILES
        pltpu.sync_copy(u_sh.at[pl.ds(s * ustripe, ustripe)],
                        uacc_out.at[pl.ds(s * ustripe, ustripe)])

    @pl.when(jnp.logical_and(c == 0, s == 0))
    def _():
        def rb(i, _):
            idx = x_v[pl.ds(i * 16, 16)]
            r_v[pl.ds(i * 16, 16)] = plsc.load_gather(poi_map, [idx])
            return 0
        lax.fori_loop(0, NSLOT // 16, rb, 0)
        pltpu.sync_copy(r_v, r_out)

    @pl.when(jnp.logical_and(c == 0, s == 1))
    def _():
        def rub(i, _):
            idx = au_v[pl.ds(i * 16, 16)]
            r_v[pl.ds(i * 16, 16)] = plsc.load_gather(usr_map, [idx])
            return 0
        lax.fori_loop(0, B // 16, rub, 0)
        pltpu.sync_copy(r_v.at[pl.ds(0, B)], ru_out)

    @pl.when(jnp.logical_and(c == 1, s == 0))
    def _():
        pltpu.async_copy(usr_hbm.at[au_v], rows_v, sem).wait()
        pltpu.sync_copy(rows_v, pu_out)


@jax.jit
def _sc_stage(x_flat, au, e1s, e1d, e1w, eiu, eip, eiw, enc, usr):
    mesh = plsc.VectorSubcoreMesh(core_axis_name="c", subcore_axis_name="s")
    out_type = (
        jax.ShapeDtypeStruct((ACC_ROWS, H), _f32),
        jax.ShapeDtypeStruct((ACC_ROWS, H), _f32),
        jax.ShapeDtypeStruct((UACC_ROWS, H), _f32),
        jax.ShapeDtypeStruct((NSLOT,), _i32),
        jax.ShapeDtypeStruct((B,), _i32),
        jax.ShapeDtypeStruct((B, H), _f32),
    )
    scratch = [
        pltpu.VMEM((P_TOT,), _i32),
        pltpu.VMEM((5008,), _i32),
        pltpu.VMEM((NSLOT,), _i32),
        pltpu.VMEM((B,), _i32),
        pltpu.VMEM((STAGE1,), _i32),
        pltpu.VMEM((STAGE1,), _i32),
        pltpu.VMEM((STAGE1,), _f32),
        pltpu.VMEM((CSBUF,), _i32),
        pltpu.VMEM((CSBUF,), _f32),
        pltpu.VMEM((CSBUF,), _i32),
        pltpu.VMEM((1, 128), _i32),
        pltpu.VMEM((128, H), _f32),
        pltpu.VMEM((NSLOT,), _i32),
        pltpu.VMEM_SHARED((ACC_ROWS, H), _f32),
        pltpu.VMEM_SHARED((ACC_ROWS, H), _f32),
        pltpu.VMEM_SHARED((UACC_ROWS, H), _f32),
        pltpu.SemaphoreType.DMA,
    ]
    return pl.kernel(_sc_body, out_type=out_type, mesh=mesh,
                     scratch_types=scratch,
                     compiler_params=pltpu.CompilerParams(
                         needs_layout_passes=False))(
        x_flat, au, e1s, e1d, e1w, eiu, eip, eiw, enc, usr)


def _tc1_body(enc_ref, poi_ref, uacc_ref, r_ref, ru_ref, pu_ref, out_ref,
              num_ref, den_ref):
    s = pl.program_id(0)
    rb = r_ref[0, 0]
    a = (rb[:, None] == lax.broadcasted_iota(_i32, (B, ACC_ROWS), 1))
    a = a.astype(_f32)
    xe = (jnp.dot(a, enc_ref[...], preferred_element_type=_f32) +
          jnp.dot(a, poi_ref[...], preferred_element_type=_f32)) * 0.5
    au = (ru_ref[0][:, None] ==
          lax.broadcasted_iota(_i32, (B, UACC_ROWS), 1)).astype(_f32)
    up = jnp.dot(au, uacc_ref[...], preferred_element_type=_f32)
    diff = up - xe
    ss = jnp.sum(diff * diff, axis=1)
    w = jnp.exp(-jnp.sqrt(ss + 1e-12)) + 1e-10
    wb = w[:, None]

    @pl.when(s == 0)
    def _():
        num_ref[...] = jnp.zeros_like(num_ref)
        den_ref[...] = jnp.zeros_like(den_ref)

    num_ref[...] += wb * xe
    den_ref[...] += jnp.broadcast_to(wb, (B, H))
    out_w = num_ref[...] / den_ref[...]
    out_ref[...] = jnp.concatenate([out_w, pu_ref[...]], axis=-1)[None]


@jax.jit
def _tc_stage1(enc_acc, poi_acc, uacc, r2d, ru2d, pu):
    return pl.pallas_call(
        _tc1_body,
        grid=(S,),
        in_specs=[
            pl.BlockSpec((ACC_ROWS, H), lambda s: (0, 0)),
            pl.BlockSpec((ACC_ROWS, H), lambda s: (0, 0)),
            pl.BlockSpec((UACC_ROWS, H), lambda s: (0, 0)),
            pl.BlockSpec((1, 1, B), lambda s: (s, 0, 0)),
            pl.BlockSpec((1, B), lambda s: (0, 0)),
            pl.BlockSpec((B, H), lambda s: (0, 0)),
        ],
        out_specs=pl.BlockSpec((1, B, 2 * H), lambda s: (s, 0, 0)),
        out_shape=jax.ShapeDtypeStruct((S, B, 2 * H), _f32),
        scratch_shapes=[pltpu.VMEM((B, H), _f32), pltpu.VMEM((B, H), _f32)],
    )(enc_acc, poi_acc, uacc, r2d, ru2d, pu)


def _tc2_body(pu_ref, w_ref, b_ref, out_ref):
    bt, pt = out_ref.shape[0], out_ref.shape[2]
    lhs = jnp.swapaxes(pu_ref[...], 0, 1).reshape(bt * S, 2 * H)
    y = jnp.dot(lhs, w_ref[...], preferred_element_type=_f32) + b_ref[...]
    out_ref[...] = y.reshape(bt, S, pt)


@jax.jit
def _tc_stage2(out_pu, fc_W, fc_b2d):
    BT, PT = 64, 1024
    return pl.pallas_call(
        _tc2_body,
        grid=(B // BT, pl.cdiv(P_TOT, PT)),
        in_specs=[
            pl.BlockSpec((S, BT, 2 * H), lambda b, p: (0, b, 0)),
            pl.BlockSpec((2 * H, PT), lambda b, p: (0, p)),
            pl.BlockSpec((1, PT), lambda b, p: (0, p)),
        ],
        out_specs=pl.BlockSpec((BT, S, PT), lambda b, p: (b, 0, p)),
        out_shape=jax.ShapeDtypeStruct((B, S, P_TOT), _f32),
    )(out_pu, fc_W, fc_b2d)


def kernel(x, active_user, edge_index, edge_weight, interact_user_idx,
           interact_poi_idx, interact_weight, enc_table, user_table,
           fc_W, fc_b):
    x_flat = x.reshape(-1)
    e1s = jnp.pad(edge_index[0], (0, E1_PAD - N_EDGES))
    e1d = jnp.pad(edge_index[1], (0, E1_PAD - N_EDGES))
    e1w = jnp.pad(edge_weight, (0, E1_PAD - N_EDGES))
    eiu = jnp.pad(interact_user_idx, (0, EI_PAD - N_INTER))
    eip = jnp.pad(interact_poi_idx, (0, EI_PAD - N_INTER))
    eiw = jnp.pad(interact_weight, (0, EI_PAD - N_INTER))

    enc_acc, poi_acc, uacc, r, ru, pu = _sc_stage(
        x_flat, active_user, e1s, e1d, e1w, eiu, eip, eiw,
        enc_table, user_table)

    out_pu = _tc_stage1(enc_acc, poi_acc, uacc, r.reshape(S, 1, B),
                        ru.reshape(1, B), pu)
    return _tc_stage2(out_pu, fc_W, fc_b.reshape(1, P_TOT))
